# baseline (speedup 1.0000x reference)
import math

import jax
import jax.numpy as jnp
from jax.experimental import pallas as pl
from jax.experimental.pallas import tpu as pltpu

_B_TILE = 256
_N = 8
_HID = 128
_LAT = 64


def _fused_kernel(eps_ref, outb_ref, f6b2_ref,
                  adj2_ref, opst_ref, wfirst_ref, wmid_ref, bias_ref,
                  fcw_ref, fcb_ref, outw_ref, f6w1_ref, f6b1_ref, f6w2_ref,
                  out_ref):
    bt = adj2_ref.shape[0]

    abc = {}
    for i in range(_N - 1):
        for j in range(i + 1, _N):
            k = i * _N + j
            abc[(i, j)] = jnp.broadcast_to(adj2_ref[:, k:k + 1], (bt, _HID))

    opsn = [opst_ref[j] for j in range(_N)]

    agg0 = []
    for i in range(_N):
        acc = jnp.zeros((bt, _N), jnp.float32)
        for j in range(i + 1, _N):
            acc = acc + adj2_ref[:, i * _N + j:i * _N + j + 1] * opsn[j]
        agg0.append(acc)

    def agg_big(xs):
        out = []
        for i in range(_N):
            acc = None
            for j in range(i + 1, _N):
                t = abc[(i, j)] * xs[j]
                acc = t if acc is None else acc + t
            out.append(acc if acc is not None
                       else jnp.zeros((bt, _HID), jnp.float32))
        return out

    def mm(a, w):
        return jax.lax.dot_general(a.astype(jnp.bfloat16), w,
                                   (((1,), (0,)), ((), ())),
                                   preferred_element_type=jnp.float32)

    logits = []
    for e in range(6):
        c0 = 1.0 + eps_ref[e, 0]
        Z = jnp.concatenate([agg0[i] + c0 * opsn[i] for i in range(_N)],
                            axis=0)
        H = jnp.maximum(mm(Z, wfirst_ref[e]) + bias_ref[e, 0], 0.0)
        H = jnp.maximum(mm(H, wmid_ref[e, 0]) + bias_ref[e, 1], 0.0)
        for l in (1, 2):
            cl = 1.0 + eps_ref[e, l]
            xs = [H[i * bt:(i + 1) * bt] for i in range(_N)]
            ag = agg_big(xs)
            Z = jnp.concatenate([ag[i] + cl * xs[i] for i in range(_N)],
                                axis=0)
            H = jnp.maximum(mm(Z, wmid_ref[e, 2 * l - 1]) + bias_ref[e, 2 * l],
                            0.0)
            H = jnp.maximum(mm(H, wmid_ref[e, 2 * l]) + bias_ref[e, 2 * l + 1],
                            0.0)
        g = H[0:bt]
        for i in range(1, _N):
            g = g + H[i * bt:(i + 1) * bt]
        lat = jnp.maximum(mm(g, fcw_ref[e]) + fcb_ref[e], 0.0)
        if e < 5:
            logits.append(mm(lat, outw_ref[e]) + outb_ref[e])
        else:
            h6 = jnp.maximum(mm(lat, f6w1_ref[...]) + f6b1_ref[...], 0.0)
            logits.append(mm(h6, f6w2_ref[...]) + f6b2_ref[0])
    out_ref[...] = jax.nn.sigmoid(jnp.concatenate(logits, axis=1))


def kernel(adj, ops, params):
    B = adj.shape[0]
    adj2 = adj.reshape(B, _N * _N)
    ops_t = jnp.pad(ops, ((0, 0), (0, 0), (0, 1))).transpose(1, 0, 2)
    bn_s = 1.0 / math.sqrt(1.0 + 1e-5)

    experts = list(params['experts']) + [params['expert6']]
    wfirst, wmid, bias, eps, fcw, fcb, outw, outb = ([] for _ in range(8))
    for p in experts:
        mats, bs = [], []
        for layer in p['convs']:
            for lin in layer['mlp']:
                sc = lin['gamma'] * bn_s
                mats.append(lin['W'] * sc[None, :])
                bs.append(lin['b'] * sc + lin['beta'])
        eps.append(jnp.stack([layer['eps'] for layer in p['convs']]))
        wfirst.append(jnp.pad(mats[0], ((0, 1), (0, 0))))
        wmid.append(jnp.stack(mats[1:]))
        bias.append(jnp.stack(bs))
        fcw.append(p['fc_W'])
        fcb.append(p['fc_b'])
        outw.append(p['out_W'])
        outb.append(p['out_b'][0])
    bf = jnp.bfloat16
    wfirst = jnp.stack(wfirst).astype(bf)
    wmid = jnp.stack(wmid).astype(bf)
    bias = jnp.stack(bias)
    eps = jnp.stack(eps)
    fcw = jnp.stack(fcw).astype(bf)
    fcb = jnp.stack(fcb)
    outw = jnp.stack(outw).astype(bf)
    outb = jnp.stack(outb)

    f = params['fc6']
    sc6 = f['gamma'] * bn_s
    f6w1 = (f['W1'] * sc6[None, :]).astype(bf)
    f6b1 = f['b1'] * sc6 + f['beta']
    f6w2 = f['W2'].astype(bf)
    f6b2 = f['b2']

    bt = _B_TILE
    grid = (B // bt,)
    smem = lambda: pl.BlockSpec(memory_space=pltpu.SMEM)
    rep = lambda shp: pl.BlockSpec(shp, lambda i: (0,) * len(shp))

    out = pl.pallas_call(
        _fused_kernel,
        grid=grid,
        in_specs=[
            smem(),
            smem(),
            smem(),
            pl.BlockSpec((bt, _N * _N), lambda i: (i, 0)),
            pl.BlockSpec((_N, bt, _N), lambda i: (0, i, 0)),
            rep((6, _N, _HID)),
            rep((6, 5, _HID, _HID)),
            rep((6, 6, _HID)),
            rep((6, _HID, _LAT)),
            rep((6, _LAT)),
            rep((6, _LAT, 1)),
            rep((_LAT, _HID)),
            rep((_HID,)),
            rep((_HID, 1)),
        ],
        out_specs=pl.BlockSpec((bt, 6), lambda i: (i, 0)),
        out_shape=jax.ShapeDtypeStruct((B, 6), jnp.float32),
        compiler_params=pltpu.CompilerParams(
            dimension_semantics=("parallel",),
        ),
    )(eps, outb, f6b2, adj2, ops_t, wfirst, wmid, bias, fcw, fcb, outw,
      f6w1, f6b1, f6w2)
    return out


# speedup vs baseline: 1.2167x; 1.2167x over previous
import math

import jax
import jax.numpy as jnp
from jax.experimental import pallas as pl
from jax.experimental.pallas import tpu as pltpu

_B_TILE = 512
_N = 8
_HID = 128
_LAT = 64


def _fused_kernel(eps_ref, outb_ref, f6b2_ref,
                  adj2_ref, opst_ref, wfirst_ref, wmid_ref, bias_ref,
                  fcw_ref, fcb_ref, outw_ref, f6w1_ref, f6b1_ref, f6w2_ref,
                  out_ref):
    bt = adj2_ref.shape[0]

    abc = {}
    for i in range(_N - 1):
        for j in range(i + 1, _N):
            k = i * _N + j
            abc[(i, j)] = jnp.broadcast_to(adj2_ref[:, k:k + 1], (bt, _HID))

    opsn = [opst_ref[j] for j in range(_N)]

    agg0 = []
    for i in range(_N):
        acc = jnp.zeros((bt, _N), jnp.float32)
        for j in range(i + 1, _N):
            acc = acc + adj2_ref[:, i * _N + j:i * _N + j + 1] * opsn[j]
        agg0.append(acc)

    def agg_big(xs):
        out = []
        for i in range(_N):
            acc = None
            for j in range(i + 1, _N):
                t = abc[(i, j)] * xs[j]
                acc = t if acc is None else acc + t
            out.append(acc if acc is not None
                       else jnp.zeros((bt, _HID), jnp.float32))
        return out

    def mm(a, w):
        return jax.lax.dot_general(a, w, (((1,), (0,)), ((), ())),
                                   preferred_element_type=jnp.float32)

    logits = []
    for e in range(6):
        c0 = 1.0 + eps_ref[e, 0]
        Z = jnp.concatenate([agg0[i] + c0 * opsn[i] for i in range(_N)],
                            axis=0)
        H = jnp.maximum(mm(Z, wfirst_ref[e]) + bias_ref[e, 0], 0.0)
        H = jnp.maximum(mm(H, wmid_ref[e, 0]) + bias_ref[e, 1], 0.0)
        for l in (1, 2):
            cl = 1.0 + eps_ref[e, l]
            xs = [H[i * bt:(i + 1) * bt] for i in range(_N)]
            ag = agg_big(xs)
            Z = jnp.concatenate([ag[i] + cl * xs[i] for i in range(_N)],
                                axis=0)
            H = jnp.maximum(mm(Z, wmid_ref[e, 2 * l - 1]) + bias_ref[e, 2 * l],
                            0.0)
            H = jnp.maximum(mm(H, wmid_ref[e, 2 * l]) + bias_ref[e, 2 * l + 1],
                            0.0)
        g = H[0:bt]
        for i in range(1, _N):
            g = g + H[i * bt:(i + 1) * bt]
        lat = jnp.maximum(mm(g, fcw_ref[e]) + fcb_ref[e], 0.0)
        if e < 5:
            logits.append(mm(lat, outw_ref[e]) + outb_ref[e])
        else:
            h6 = jnp.maximum(mm(lat, f6w1_ref[...]) + f6b1_ref[...], 0.0)
            logits.append(mm(h6, f6w2_ref[...]) + f6b2_ref[0])
    out_ref[...] = jax.nn.sigmoid(jnp.concatenate(logits, axis=1))


def kernel(adj, ops, params):
    B = adj.shape[0]
    adj2 = adj.reshape(B, _N * _N)
    ops_t = jnp.pad(ops, ((0, 0), (0, 0), (0, 1))).transpose(1, 0, 2)
    bn_s = 1.0 / math.sqrt(1.0 + 1e-5)

    experts = list(params['experts']) + [params['expert6']]
    wfirst, wmid, bias, eps, fcw, fcb, outw, outb = ([] for _ in range(8))
    for p in experts:
        mats, bs = [], []
        for layer in p['convs']:
            for lin in layer['mlp']:
                sc = lin['gamma'] * bn_s
                mats.append(lin['W'] * sc[None, :])
                bs.append(lin['b'] * sc + lin['beta'])
        eps.append(jnp.stack([layer['eps'] for layer in p['convs']]))
        wfirst.append(jnp.pad(mats[0], ((0, 1), (0, 0))))
        wmid.append(jnp.stack(mats[1:]))
        bias.append(jnp.stack(bs))
        fcw.append(p['fc_W'])
        fcb.append(p['fc_b'])
        outw.append(p['out_W'])
        outb.append(p['out_b'][0])
    wfirst = jnp.stack(wfirst)
    wmid = jnp.stack(wmid)
    bias = jnp.stack(bias)
    eps = jnp.stack(eps)
    fcw = jnp.stack(fcw)
    fcb = jnp.stack(fcb)
    outw = jnp.stack(outw)
    outb = jnp.stack(outb)

    f = params['fc6']
    sc6 = f['gamma'] * bn_s
    f6w1 = f['W1'] * sc6[None, :]
    f6b1 = f['b1'] * sc6 + f['beta']
    f6w2 = f['W2']
    f6b2 = f['b2']

    bt = _B_TILE
    grid = (B // bt,)
    smem = lambda: pl.BlockSpec(memory_space=pltpu.SMEM)
    rep = lambda shp: pl.BlockSpec(shp, lambda i: (0,) * len(shp))

    out = pl.pallas_call(
        _fused_kernel,
        grid=grid,
        in_specs=[
            smem(),
            smem(),
            smem(),
            pl.BlockSpec((bt, _N * _N), lambda i: (i, 0)),
            pl.BlockSpec((_N, bt, _N), lambda i: (0, i, 0)),
            rep((6, _N, _HID)),
            rep((6, 5, _HID, _HID)),
            rep((6, 6, _HID)),
            rep((6, _HID, _LAT)),
            rep((6, _LAT)),
            rep((6, _LAT, 1)),
            rep((_LAT, _HID)),
            rep((_HID,)),
            rep((_HID, 1)),
        ],
        out_specs=pl.BlockSpec((bt, 6), lambda i: (i, 0)),
        out_shape=jax.ShapeDtypeStruct((B, 6), jnp.float32),
        compiler_params=pltpu.CompilerParams(
            dimension_semantics=("parallel",),
        ),
    )(eps, outb, f6b2, adj2, ops_t, wfirst, wmid, bias, fcw, fcb, outw,
      f6w1, f6b1, f6w2)
    return out
